# baseline (speedup 1.0000x reference)
import functools

import jax
import jax.numpy as jnp
from jax import lax
from jax.experimental import pallas as pl
from jax.experimental.pallas import tpu as pltpu
from jax.experimental.pallas import tpu_sc as plsc

_N = 20000
_K = 5000
_T = 512
_KP = 5120
_NSTRIP = _KP // _T
_THR = 0.5
_NCOL = 7


_NW = 32
_PER_W = _KP // _NW

_sc_mesh = plsc.VectorSubcoreMesh(core_axis_name="c", subcore_axis_name="s")


@functools.partial(
    pl.kernel,
    mesh=_sc_mesh,
    out_type=tuple(
        jax.ShapeDtypeStruct((_KP,), jnp.float32) for _ in range(_NCOL)),
    scratch_types=[pltpu.VMEM((_PER_W,), jnp.int32)]
    + [pltpu.VMEM((_PER_W,), jnp.float32) for _ in range(_NCOL)]
    + [pltpu.SemaphoreType.DMA],
)
def _sc_gather(*refs):
    idx_hbm = refs[0]
    cols_hbm = refs[1:1 + _NCOL]
    outs_hbm = refs[1 + _NCOL:1 + 2 * _NCOL]
    idx_v = refs[1 + 2 * _NCOL]
    cols_v = refs[2 + 2 * _NCOL:2 + 3 * _NCOL]
    sem = refs[2 + 3 * _NCOL]
    wid = lax.axis_index("s") * 2 + lax.axis_index("c")
    base = wid * _PER_W
    pltpu.sync_copy(idx_hbm.at[pl.ds(base, _PER_W)], idx_v)
    handles = [
        pltpu.async_copy(cols_hbm[c].at[idx_v], cols_v[c], sem)
        for c in range(_NCOL)
    ]
    for h in handles:
        h.wait()
    for c in range(_NCOL):
        pltpu.sync_copy(cols_v[c], outs_hbm[c].at[pl.ds(base, _PER_W)])


def _nms_body(cxr_ref, cyr_ref, czr_ref, wr_ref, lr_ref, hr_ref, sr_ref,
              cxc_ref, cyc_ref, wc_ref, lc_ref, out_ref):
    i = pl.program_id(0)
    cxr = cxr_ref[...]
    cyr = cyr_ref[...]
    wr = wr_ref[...]
    lr = lr_ref[...]
    x1r = cxr - wr * 0.5
    x2r = cxr + wr * 0.5
    y1r = cyr - lr * 0.5
    y2r = cyr + lr * 0.5
    ar2 = wr * lr * _THR

    def tile_s(j):
        sl = pl.ds(j * _T, _T)
        cxc = cxc_ref[:, sl]
        cyc = cyc_ref[:, sl]
        wc = wc_ref[:, sl]
        lc = lc_ref[:, sl]
        x1c = cxc - wc * 0.5
        x2c = cxc + wc * 0.5
        y1c = cyc - lc * 0.5
        y2c = cyc + lc * 0.5
        ac2 = wc * lc * _THR
        ix = jnp.maximum(
            jnp.minimum(x2r, x2c) - jnp.maximum(x1r, x1c), 0.0)
        iy = jnp.maximum(
            jnp.minimum(y2r, y2c) - jnp.maximum(y1r, y1c), 0.0)
        inter = ix * iy
        return (1.0 + _THR) * inter - (ar2 + ac2)

    def body(j, m):
        return jnp.maximum(m, tile_s(j))

    m = lax.fori_loop(0, i, body, jnp.full((_T, _T), -1.0, jnp.float32))
    sd = tile_s(i)
    row_ids = lax.broadcasted_iota(jnp.int32, (_T, 1), 0)
    col_ids = lax.broadcasted_iota(jnp.int32, (1, _T), 1)
    m = jnp.maximum(m, jnp.where(col_ids < row_ids, sd, -1.0))
    keep = (jnp.max(m, axis=1, keepdims=True) <= 0.0).astype(jnp.float32)
    out_ref[:, 0:1] = cxr * keep
    out_ref[:, 1:2] = cyr * keep
    out_ref[:, 2:3] = czr_ref[...] * keep
    out_ref[:, 3:4] = wr * keep
    out_ref[:, 4:5] = lr * keep
    out_ref[:, 5:6] = hr_ref[...] * keep
    out_ref[:, 6:7] = sr_ref[...] * keep
    out_ref[:, 7:8] = jnp.zeros((_T, 1), jnp.float32)


_row_spec = pl.BlockSpec((_T, 1), lambda i: (i, 0))
_col_spec = pl.BlockSpec((1, _KP), lambda i: (0, 0))

_nms_call = pl.pallas_call(
    _nms_body,
    grid=(_NSTRIP,),
    in_specs=[_row_spec] * 7 + [_col_spec] * 4,
    out_specs=pl.BlockSpec((_T, 8), lambda i: (i, 0)),
    out_shape=jax.ShapeDtypeStruct((_KP, 8), jnp.float32),
)


@jax.jit
def kernel(boxes, scores):
    _, top_idx = lax.top_k(scores, _KP)
    cols = [boxes[:, c] for c in range(6)] + [scores]
    g = _sc_gather(top_idx, *cols)
    rows = [v[:, None] for v in g]
    cvecs = [g[c][None, :] for c in (0, 1, 3, 4)]
    out = _nms_call(*rows, *cvecs)
    return out[:_K, :7]


# speedup vs baseline: 1.1474x; 1.1474x over previous
import functools

import jax
import jax.numpy as jnp
from jax import lax
from jax.experimental import pallas as pl
from jax.experimental.pallas import tpu as pltpu
from jax.experimental.pallas import tpu_sc as plsc

_N = 20000
_K = 5000
_T = 512
_KP = 5120
_NSTRIP = _KP // _T
_THR = 0.5
_NCOL = 7


_NW = 32
_PER_W = _KP // _NW

_sc_mesh = plsc.VectorSubcoreMesh(core_axis_name="c", subcore_axis_name="s")


@functools.partial(
    pl.kernel,
    mesh=_sc_mesh,
    out_type=tuple(
        jax.ShapeDtypeStruct((_KP,), jnp.float32) for _ in range(_NCOL)),
    scratch_types=[pltpu.VMEM((_PER_W,), jnp.int32)]
    + [pltpu.VMEM((_PER_W,), jnp.float32) for _ in range(_NCOL)]
    + [pltpu.SemaphoreType.DMA],
)
def _sc_gather(*refs):
    idx_hbm = refs[0]
    cols_hbm = refs[1:1 + _NCOL]
    outs_hbm = refs[1 + _NCOL:1 + 2 * _NCOL]
    idx_v = refs[1 + 2 * _NCOL]
    cols_v = refs[2 + 2 * _NCOL:2 + 3 * _NCOL]
    sem = refs[2 + 3 * _NCOL]
    wid = lax.axis_index("s") * 2 + lax.axis_index("c")
    base = wid * _PER_W
    pltpu.sync_copy(idx_hbm.at[pl.ds(base, _PER_W)], idx_v)
    handles = [
        pltpu.async_copy(cols_hbm[c].at[idx_v], cols_v[c], sem)
        for c in range(_NCOL)
    ]
    for h in handles:
        h.wait()
    for c in range(_NCOL):
        pltpu.sync_copy(cols_v[c], outs_hbm[c].at[pl.ds(base, _PER_W)])


def _nms_body(cxc_ref, cyc_ref, czc_ref, wc_ref, lc_ref, hc_ref, sc_ref,
              out_ref):
    i = pl.program_id(0)
    me = pl.ds(i * _T, _T)

    def rowv(ref):
        return jnp.transpose(ref[:, me], (1, 0))

    cxr = rowv(cxc_ref)
    cyr = rowv(cyc_ref)
    wr = rowv(wc_ref)
    lr = rowv(lc_ref)
    x1r = cxr - wr * 0.5
    x2r = cxr + wr * 0.5
    y1r = cyr - lr * 0.5
    y2r = cyr + lr * 0.5
    ar2 = wr * lr * _THR

    def tile_s(j):
        sl = pl.ds(j * _T, _T)
        cxc = cxc_ref[:, sl]
        cyc = cyc_ref[:, sl]
        wc = wc_ref[:, sl]
        lc = lc_ref[:, sl]
        x1c = cxc - wc * 0.5
        x2c = cxc + wc * 0.5
        y1c = cyc - lc * 0.5
        y2c = cyc + lc * 0.5
        ac2 = wc * lc * _THR
        ix = jnp.maximum(
            jnp.minimum(x2r, x2c) - jnp.maximum(x1r, x1c), 0.0)
        iy = jnp.maximum(
            jnp.minimum(y2r, y2c) - jnp.maximum(y1r, y1c), 0.0)
        inter = ix * iy
        return (1.0 + _THR) * inter - (ar2 + ac2)

    def body(j, m):
        return jnp.maximum(m, tile_s(j))

    m = lax.fori_loop(0, i, body, jnp.full((_T, _T), -1.0, jnp.float32))
    sd = tile_s(i)
    row_ids = lax.broadcasted_iota(jnp.int32, (_T, 1), 0)
    col_ids = lax.broadcasted_iota(jnp.int32, (1, _T), 1)
    m = jnp.maximum(m, jnp.where(col_ids < row_ids, sd, -1.0))
    keep = (jnp.max(m, axis=1, keepdims=True) <= 0.0).astype(jnp.float32)
    out_ref[:, 0:1] = cxr * keep
    out_ref[:, 1:2] = cyr * keep
    out_ref[:, 2:3] = rowv(czc_ref) * keep
    out_ref[:, 3:4] = wr * keep
    out_ref[:, 4:5] = lr * keep
    out_ref[:, 5:6] = rowv(hc_ref) * keep
    out_ref[:, 6:7] = rowv(sc_ref) * keep
    out_ref[:, 7:8] = jnp.zeros((_T, 1), jnp.float32)


_col_spec = pl.BlockSpec((1, _KP), lambda i: (0, 0))

_nms_call = pl.pallas_call(
    _nms_body,
    grid=(_NSTRIP,),
    in_specs=[_col_spec] * 7,
    out_specs=pl.BlockSpec((_T, 8), lambda i: (i, 0)),
    out_shape=jax.ShapeDtypeStruct((_KP, 8), jnp.float32),
)


@jax.jit
def kernel(boxes, scores):
    _, top_idx = lax.top_k(scores, _KP)
    boxes_t = boxes.T
    cols = [boxes_t[c] for c in range(6)] + [scores]
    g = _sc_gather(top_idx, *cols)
    cvecs = [v[None, :] for v in g]
    out = _nms_call(*cvecs)
    return out[:_K, :7]


# speedup vs baseline: 1.2405x; 1.0812x over previous
import functools

import jax
import jax.numpy as jnp
from jax import lax
from jax.experimental import pallas as pl
from jax.experimental.pallas import tpu as pltpu
from jax.experimental.pallas import tpu_sc as plsc

_N = 20000
_K = 5000
_T = 512
_KP = 5120
_NSTRIP = _KP // _T
_THR = 0.5
_NCOL = 7


_NW = 32
_PER_W = _KP // _NW

_sc_mesh = plsc.VectorSubcoreMesh(core_axis_name="c", subcore_axis_name="s")


@functools.partial(
    pl.kernel,
    mesh=_sc_mesh,
    out_type=tuple(
        jax.ShapeDtypeStruct((_KP,), jnp.float32) for _ in range(_NCOL)),
    scratch_types=[pltpu.VMEM((_PER_W,), jnp.int32)]
    + [pltpu.VMEM((_PER_W,), jnp.float32) for _ in range(_NCOL)]
    + [pltpu.SemaphoreType.DMA],
)
def _sc_gather(*refs):
    idx_hbm = refs[0]
    cols_hbm = refs[1:1 + _NCOL]
    outs_hbm = refs[1 + _NCOL:1 + 2 * _NCOL]
    idx_v = refs[1 + 2 * _NCOL]
    cols_v = refs[2 + 2 * _NCOL:2 + 3 * _NCOL]
    sem = refs[2 + 3 * _NCOL]
    wid = lax.axis_index("s") * 2 + lax.axis_index("c")
    base = wid * _PER_W
    pltpu.sync_copy(idx_hbm.at[pl.ds(base, _PER_W)], idx_v)
    handles = [
        pltpu.async_copy(cols_hbm[c].at[idx_v], cols_v[c], sem)
        for c in range(_NCOL)
    ]
    for h in handles:
        h.wait()
    for c in range(_NCOL):
        pltpu.sync_copy(cols_v[c], outs_hbm[c].at[pl.ds(base, _PER_W)])


def _nms_body(cxc_ref, cyc_ref, czc_ref, wc_ref, lc_ref, hc_ref, sc_ref,
              out_ref):
    i = pl.program_id(0)
    me = pl.ds(i * _T, _T)

    def rowv(ref):
        return jnp.transpose(ref[:, me], (1, 0))

    cxr = rowv(cxc_ref)
    cyr = rowv(cyc_ref)
    wr = rowv(wc_ref)
    lr = rowv(lc_ref)
    x1r = cxr - wr * 0.5
    x2r = cxr + wr * 0.5
    y1r = cyr - lr * 0.5
    y2r = cyr + lr * 0.5
    ar2 = wr * lr * _THR

    def tile_s(j):
        sl = pl.ds(j * _T, _T)
        cxc = cxc_ref[:, sl]
        cyc = cyc_ref[:, sl]
        wc = wc_ref[:, sl]
        lc = lc_ref[:, sl]
        x1c = cxc - wc * 0.5
        x2c = cxc + wc * 0.5
        y1c = cyc - lc * 0.5
        y2c = cyc + lc * 0.5
        ac2 = wc * lc * _THR
        ix = jnp.maximum(
            jnp.minimum(x2r, x2c) - jnp.maximum(x1r, x1c), 0.0)
        iy = jnp.maximum(
            jnp.minimum(y2r, y2c) - jnp.maximum(y1r, y1c), 0.0)
        inter = ix * iy
        return (1.0 + _THR) * inter - (ar2 + ac2)

    def lane_fold(s):
        return jnp.maximum(
            jnp.maximum(s[:, 0:128], s[:, 128:256]),
            jnp.maximum(s[:, 256:384], s[:, 384:512]))

    def body(j, m):
        return jnp.maximum(m, lane_fold(tile_s(j)))

    m = lax.fori_loop(0, i, body, jnp.full((_T, 128), -1.0, jnp.float32))
    sd = tile_s(i)
    row_ids = lax.broadcasted_iota(jnp.int32, (_T, 1), 0)
    col_ids = lax.broadcasted_iota(jnp.int32, (1, _T), 1)
    m = jnp.maximum(m, lane_fold(jnp.where(col_ids < row_ids, sd, -1.0)))
    keep = (jnp.max(m, axis=1, keepdims=True) <= 0.0).astype(jnp.float32)
    row8 = jnp.concatenate(
        [cxr, cyr, rowv(czc_ref), wr, lr, rowv(hc_ref), rowv(sc_ref),
         jnp.zeros((_T, 1), jnp.float32)], axis=1)
    out_ref[...] = row8 * keep


_col_spec = pl.BlockSpec((1, _KP), lambda i: (0, 0))

_nms_call = pl.pallas_call(
    _nms_body,
    grid=(_NSTRIP,),
    in_specs=[_col_spec] * 7,
    out_specs=pl.BlockSpec((_T, 8), lambda i: (i, 0)),
    out_shape=jax.ShapeDtypeStruct((_KP, 8), jnp.float32),
)


@jax.jit
def kernel(boxes, scores):
    _, top_idx = lax.top_k(scores, _KP)
    boxes_t = boxes.T
    cols = [boxes_t[c] for c in range(6)] + [scores]
    g = _sc_gather(top_idx, *cols)
    cvecs = [v[None, :] for v in g]
    out = _nms_call(*cvecs)
    return out[:_K, :7]


# speedup vs baseline: 1.3372x; 1.0779x over previous
import functools

import jax
import jax.numpy as jnp
from jax import lax
from jax.experimental import pallas as pl
from jax.experimental.pallas import tpu as pltpu
from jax.experimental.pallas import tpu_sc as plsc

_N = 20000
_K = 5000
_T = 1024
_KP = 5120
_NSTRIP = _KP // _T
_THR = 0.5
_NCOL = 7


_NW = 32
_PER_W = _KP // _NW

_sc_mesh = plsc.VectorSubcoreMesh(core_axis_name="c", subcore_axis_name="s")


@functools.partial(
    pl.kernel,
    mesh=_sc_mesh,
    out_type=tuple(
        jax.ShapeDtypeStruct((_KP,), jnp.float32) for _ in range(_NCOL)),
    scratch_types=[pltpu.VMEM((_PER_W,), jnp.int32)]
    + [pltpu.VMEM((_PER_W,), jnp.float32) for _ in range(_NCOL)]
    + [pltpu.SemaphoreType.DMA],
)
def _sc_gather(*refs):
    idx_hbm = refs[0]
    cols_hbm = refs[1:1 + _NCOL]
    outs_hbm = refs[1 + _NCOL:1 + 2 * _NCOL]
    idx_v = refs[1 + 2 * _NCOL]
    cols_v = refs[2 + 2 * _NCOL:2 + 3 * _NCOL]
    sem = refs[2 + 3 * _NCOL]
    wid = lax.axis_index("s") * 2 + lax.axis_index("c")
    base = wid * _PER_W
    pltpu.sync_copy(idx_hbm.at[pl.ds(base, _PER_W)], idx_v)
    handles = [
        pltpu.async_copy(cols_hbm[c].at[idx_v], cols_v[c], sem)
        for c in range(_NCOL)
    ]
    for h in handles:
        h.wait()
    for c in range(_NCOL):
        pltpu.sync_copy(cols_v[c], outs_hbm[c].at[pl.ds(base, _PER_W)])


def _nms_body(cxc_ref, cyc_ref, czc_ref, wc_ref, lc_ref, hc_ref, sc_ref,
              out_ref):
    i = pl.program_id(0)
    me = pl.ds(i * _T, _T)

    def rowv(ref):
        return jnp.transpose(ref[:, me], (1, 0))

    cxr = rowv(cxc_ref)
    cyr = rowv(cyc_ref)
    wr = rowv(wc_ref)
    lr = rowv(lc_ref)
    x1r = cxr - wr * 0.5
    x2r = cxr + wr * 0.5
    y1r = cyr - lr * 0.5
    y2r = cyr + lr * 0.5
    ar2 = wr * lr * _THR

    def tile_s(j):
        sl = pl.ds(j * _T, _T)
        cxc = cxc_ref[:, sl]
        cyc = cyc_ref[:, sl]
        wc = wc_ref[:, sl]
        lc = lc_ref[:, sl]
        x1c = cxc - wc * 0.5
        x2c = cxc + wc * 0.5
        y1c = cyc - lc * 0.5
        y2c = cyc + lc * 0.5
        ac2 = wc * lc * _THR
        ix = jnp.maximum(
            jnp.minimum(x2r, x2c) - jnp.maximum(x1r, x1c), 0.0)
        iy = jnp.maximum(
            jnp.minimum(y2r, y2c) - jnp.maximum(y1r, y1c), 0.0)
        inter = ix * iy
        return (1.0 + _THR) * inter - (ar2 + ac2)

    def lane_fold(s):
        m = s[:, 0:128]
        for k in range(128, _T, 128):
            m = jnp.maximum(m, s[:, k:k + 128])
        return m

    def body(j, m):
        return jnp.maximum(m, lane_fold(tile_s(j)))

    m = lax.fori_loop(0, i, body, jnp.full((_T, 128), -1.0, jnp.float32))
    sd = tile_s(i)
    row_ids = lax.broadcasted_iota(jnp.int32, (_T, 1), 0)
    col_ids = lax.broadcasted_iota(jnp.int32, (1, _T), 1)
    m = jnp.maximum(m, lane_fold(jnp.where(col_ids < row_ids, sd, -1.0)))
    keep = (jnp.max(m, axis=1, keepdims=True) <= 0.0).astype(jnp.float32)
    row8 = jnp.concatenate(
        [cxr, cyr, rowv(czc_ref), wr, lr, rowv(hc_ref), rowv(sc_ref),
         jnp.zeros((_T, 1), jnp.float32)], axis=1)
    out_ref[...] = row8 * keep


_col_spec = pl.BlockSpec((1, _KP), lambda i: (0, 0))

_nms_call = pl.pallas_call(
    _nms_body,
    grid=(_NSTRIP,),
    in_specs=[_col_spec] * 7,
    out_specs=pl.BlockSpec((_T, 8), lambda i: (i, 0)),
    out_shape=jax.ShapeDtypeStruct((_KP, 8), jnp.float32),
)


@jax.jit
def kernel(boxes, scores):
    _, top_idx = lax.top_k(scores, _KP)
    boxes_t = boxes.T
    cols = [boxes_t[c] for c in range(6)] + [scores]
    g = _sc_gather(top_idx, *cols)
    cvecs = [v[None, :] for v in g]
    out = _nms_call(*cvecs)
    return out[:_K, :7]


# speedup vs baseline: 1.3459x; 1.0065x over previous
import functools

import jax
import jax.numpy as jnp
from jax import lax
from jax.experimental import pallas as pl
from jax.experimental.pallas import tpu as pltpu
from jax.experimental.pallas import tpu_sc as plsc

_N = 20000
_K = 5000
_T = 1280
_KP = 5120
_NSTRIP = _KP // _T
_THR = 0.5
_NCOL = 7


_NW = 32
_PER_W = _KP // _NW

_sc_mesh = plsc.VectorSubcoreMesh(core_axis_name="c", subcore_axis_name="s")


@functools.partial(
    pl.kernel,
    mesh=_sc_mesh,
    out_type=tuple(
        jax.ShapeDtypeStruct((_KP,), jnp.float32) for _ in range(_NCOL)),
    scratch_types=[pltpu.VMEM((_PER_W,), jnp.int32)]
    + [pltpu.VMEM((_PER_W,), jnp.float32) for _ in range(_NCOL)]
    + [pltpu.SemaphoreType.DMA],
)
def _sc_gather(*refs):
    idx_hbm = refs[0]
    cols_hbm = refs[1:1 + _NCOL]
    outs_hbm = refs[1 + _NCOL:1 + 2 * _NCOL]
    idx_v = refs[1 + 2 * _NCOL]
    cols_v = refs[2 + 2 * _NCOL:2 + 3 * _NCOL]
    sem = refs[2 + 3 * _NCOL]
    wid = lax.axis_index("s") * 2 + lax.axis_index("c")
    base = wid * _PER_W
    pltpu.sync_copy(idx_hbm.at[pl.ds(base, _PER_W)], idx_v)
    handles = [
        pltpu.async_copy(cols_hbm[c].at[idx_v], cols_v[c], sem)
        for c in range(_NCOL)
    ]
    for h in handles:
        h.wait()
    for c in range(_NCOL):
        pltpu.sync_copy(cols_v[c], outs_hbm[c].at[pl.ds(base, _PER_W)])


def _nms_body(cxc_ref, cyc_ref, czc_ref, wc_ref, lc_ref, hc_ref, sc_ref,
              out_ref):
    i = pl.program_id(0)
    me = pl.ds(i * _T, _T)

    def rowv(ref):
        return jnp.transpose(ref[:, me], (1, 0))

    cxr = rowv(cxc_ref)
    cyr = rowv(cyc_ref)
    wr = rowv(wc_ref)
    lr = rowv(lc_ref)
    x1r = cxr - wr * 0.5
    x2r = cxr + wr * 0.5
    y1r = cyr - lr * 0.5
    y2r = cyr + lr * 0.5
    ar2 = wr * lr * _THR

    def tile_s(j):
        sl = pl.ds(j * _T, _T)
        cxc = cxc_ref[:, sl]
        cyc = cyc_ref[:, sl]
        wc = wc_ref[:, sl]
        lc = lc_ref[:, sl]
        x1c = cxc - wc * 0.5
        x2c = cxc + wc * 0.5
        y1c = cyc - lc * 0.5
        y2c = cyc + lc * 0.5
        ac2 = wc * lc * _THR
        ix = jnp.maximum(
            jnp.minimum(x2r, x2c) - jnp.maximum(x1r, x1c), 0.0)
        iy = jnp.maximum(
            jnp.minimum(y2r, y2c) - jnp.maximum(y1r, y1c), 0.0)
        inter = ix * iy
        return (1.0 + _THR) * inter - (ar2 + ac2)

    def lane_fold(s):
        m = s[:, 0:128]
        for k in range(128, _T, 128):
            m = jnp.maximum(m, s[:, k:k + 128])
        return m

    def body(j, m):
        return jnp.maximum(m, lane_fold(tile_s(j)))

    m = lax.fori_loop(0, i, body, jnp.full((_T, 128), -1.0, jnp.float32))
    sd = tile_s(i)
    row_ids = lax.broadcasted_iota(jnp.int32, (_T, 1), 0)
    col_ids = lax.broadcasted_iota(jnp.int32, (1, _T), 1)
    m = jnp.maximum(m, lane_fold(jnp.where(col_ids < row_ids, sd, -1.0)))
    keep = (jnp.max(m, axis=1, keepdims=True) <= 0.0).astype(jnp.float32)
    row8 = jnp.concatenate(
        [cxr, cyr, rowv(czc_ref), wr, lr, rowv(hc_ref), rowv(sc_ref),
         jnp.zeros((_T, 1), jnp.float32)], axis=1)
    out_ref[...] = row8 * keep


_col_spec = pl.BlockSpec((1, _KP), lambda i: (0, 0))

_nms_call = pl.pallas_call(
    _nms_body,
    grid=(_NSTRIP,),
    in_specs=[_col_spec] * 7,
    out_specs=pl.BlockSpec((_T, 8), lambda i: (i, 0)),
    out_shape=jax.ShapeDtypeStruct((_KP, 8), jnp.float32),
)


@jax.jit
def kernel(boxes, scores):
    _, top_idx = lax.top_k(scores, _KP)
    boxes_t = boxes.T
    cols = [boxes_t[c] for c in range(6)] + [scores]
    g = _sc_gather(top_idx, *cols)
    cvecs = [v[None, :] for v in g]
    out = _nms_call(*cvecs)
    return out[:_K, :7]


# speedup vs baseline: 1.3937x; 1.0355x over previous
import functools

import jax
import jax.numpy as jnp
from jax import lax
from jax.experimental import pallas as pl
from jax.experimental.pallas import tpu as pltpu
from jax.experimental.pallas import tpu_sc as plsc

_N = 20000
_K = 5000
_T = 1280
_KP = 5120
_NSTRIP = _KP // _T
_THR = 0.5
_NCOL = 7


_NW = 32
_PER_W = _KP // _NW

_sc_mesh = plsc.VectorSubcoreMesh(core_axis_name="c", subcore_axis_name="s")


@functools.partial(
    pl.kernel,
    mesh=_sc_mesh,
    out_type=tuple(
        jax.ShapeDtypeStruct((_KP,), jnp.float32) for _ in range(_NCOL)),
    scratch_types=[pltpu.VMEM((_PER_W,), jnp.int32)]
    + [pltpu.VMEM((_PER_W,), jnp.float32) for _ in range(_NCOL)]
    + [pltpu.SemaphoreType.DMA],
)
def _sc_gather(*refs):
    idx_hbm = refs[0]
    cols_hbm = refs[1:1 + _NCOL]
    outs_hbm = refs[1 + _NCOL:1 + 2 * _NCOL]
    idx_v = refs[1 + 2 * _NCOL]
    cols_v = refs[2 + 2 * _NCOL:2 + 3 * _NCOL]
    sem = refs[2 + 3 * _NCOL]
    wid = lax.axis_index("s") * 2 + lax.axis_index("c")
    base = wid * _PER_W
    pltpu.sync_copy(idx_hbm.at[pl.ds(base, _PER_W)], idx_v)
    handles = [
        pltpu.async_copy(cols_hbm[c].at[idx_v], cols_v[c], sem)
        for c in range(_NCOL)
    ]
    for h in handles:
        h.wait()
    for c in range(_NCOL):
        pltpu.sync_copy(cols_v[c], outs_hbm[c].at[pl.ds(base, _PER_W)])


def _nms_body(cxc_ref, cyc_ref, czc_ref, wc_ref, lc_ref, hc_ref, sc_ref,
              out_ref):
    i = pl.program_id(0)
    me = pl.ds(i * _T, _T)

    def rowv(ref):
        return jnp.transpose(ref[:, me], (1, 0))

    cxr = rowv(cxc_ref)
    cyr = rowv(cyc_ref)
    wr = rowv(wc_ref)
    lr = rowv(lc_ref)
    x1r = cxr - wr * 0.5
    x2r = cxr + wr * 0.5
    y1r = cyr - lr * 0.5
    y2r = cyr + lr * 0.5
    ar2 = wr * lr * (_THR / (1.0 + _THR))

    def tile_s(j):
        sl = pl.ds(j * _T, _T)
        cxc = cxc_ref[:, sl]
        cyc = cyc_ref[:, sl]
        wc = wc_ref[:, sl]
        lc = lc_ref[:, sl]
        x1c = cxc - wc * 0.5
        x2c = cxc + wc * 0.5
        y1c = cyc - lc * 0.5
        y2c = cyc + lc * 0.5
        ac2 = wc * lc * (_THR / (1.0 + _THR))
        ix = jnp.minimum(x2r, x2c) - jnp.maximum(x1r, x1c)
        iy = jnp.maximum(
            jnp.minimum(y2r, y2c) - jnp.maximum(y1r, y1c), 0.0)
        inter = ix * iy
        return inter - (ar2 + ac2)

    def lane_fold(s):
        m = s[:, 0:128]
        for k in range(128, _T, 128):
            m = jnp.maximum(m, s[:, k:k + 128])
        return m

    def body(j, m):
        return jnp.maximum(m, lane_fold(tile_s(j)))

    m = lax.fori_loop(0, i, body, jnp.full((_T, 128), -1.0, jnp.float32))
    sd = tile_s(i)
    row_ids = lax.broadcasted_iota(jnp.int32, (_T, 1), 0)
    col_ids = lax.broadcasted_iota(jnp.int32, (1, _T), 1)
    m = jnp.maximum(m, lane_fold(jnp.where(col_ids < row_ids, sd, -1.0)))
    keep = (jnp.max(m, axis=1, keepdims=True) <= 0.0).astype(jnp.float32)
    row8 = jnp.concatenate(
        [cxr, cyr, rowv(czc_ref), wr, lr, rowv(hc_ref), rowv(sc_ref),
         jnp.zeros((_T, 1), jnp.float32)], axis=1)
    out_ref[...] = row8 * keep


_col_spec = pl.BlockSpec((1, _KP), lambda i: (0, 0))

_nms_call = pl.pallas_call(
    _nms_body,
    grid=(_NSTRIP,),
    in_specs=[_col_spec] * 7,
    out_specs=pl.BlockSpec((_T, 8), lambda i: (i, 0)),
    out_shape=jax.ShapeDtypeStruct((_KP, 8), jnp.float32),
)


@jax.jit
def kernel(boxes, scores):
    _, top_idx = lax.top_k(scores, _KP)
    boxes_t = boxes.T
    cols = [boxes_t[c] for c in range(6)] + [scores]
    g = _sc_gather(top_idx, *cols)
    cvecs = [v[None, :] for v in g]
    out = _nms_call(*cvecs)
    return out[:_K, :7]


# speedup vs baseline: 1.4018x; 1.0058x over previous
import functools

import jax
import jax.numpy as jnp
from jax import lax
from jax.experimental import pallas as pl
from jax.experimental.pallas import tpu as pltpu
from jax.experimental.pallas import tpu_sc as plsc

_N = 20000
_K = 5000
_T = 1280
_KP = 5120
_NSTRIP = _KP // _T
_THR = 0.5
_NCOL = 7


_NW = 32
_PER_W = _KP // _NW

_sc_mesh = plsc.VectorSubcoreMesh(core_axis_name="c", subcore_axis_name="s")


@functools.partial(
    pl.kernel,
    mesh=_sc_mesh,
    out_type=tuple(
        jax.ShapeDtypeStruct((_KP,), jnp.float32) for _ in range(_NCOL)),
    scratch_types=[pltpu.VMEM((_PER_W,), jnp.int32)]
    + [pltpu.VMEM((_PER_W,), jnp.float32) for _ in range(_NCOL)]
    + [pltpu.SemaphoreType.DMA],
)
def _sc_gather(*refs):
    idx_hbm = refs[0]
    cols_hbm = refs[1:1 + _NCOL]
    outs_hbm = refs[1 + _NCOL:1 + 2 * _NCOL]
    idx_v = refs[1 + 2 * _NCOL]
    cols_v = refs[2 + 2 * _NCOL:2 + 3 * _NCOL]
    sem = refs[2 + 3 * _NCOL]
    wid = lax.axis_index("s") * 2 + lax.axis_index("c")
    base = wid * _PER_W
    pltpu.sync_copy(idx_hbm.at[pl.ds(base, _PER_W)], idx_v)
    handles = [
        pltpu.async_copy(cols_hbm[c].at[idx_v], cols_v[c], sem)
        for c in range(_NCOL)
    ]
    for h in handles:
        h.wait()
    for c in range(_NCOL):
        pltpu.sync_copy(cols_v[c], outs_hbm[c].at[pl.ds(base, _PER_W)])


def _nms_body(cxc_ref, cyc_ref, czc_ref, wc_ref, lc_ref, hc_ref, sc_ref,
              out_ref):
    i = pl.program_id(0)
    me = pl.ds(i * _T, _T)

    def rowv(ref):
        return jnp.transpose(ref[:, me], (1, 0))

    cxr = rowv(cxc_ref)
    cyr = rowv(cyc_ref)
    wr = rowv(wc_ref)
    lr = rowv(lc_ref)
    x1r = cxr - wr * 0.5
    x2r = cxr + wr * 0.5
    y1r = cyr - lr * 0.5
    y2r = cyr + lr * 0.5
    ar2 = wr * lr * (_THR / (1.0 + _THR))

    def tile_s(csl, r0, r1, w):
        cxc = cxc_ref[:, csl]
        cyc = cyc_ref[:, csl]
        wc = wc_ref[:, csl]
        lc = lc_ref[:, csl]
        x1c = cxc - wc * 0.5
        x2c = cxc + wc * 0.5
        y1c = cyc - lc * 0.5
        y2c = cyc + lc * 0.5
        ac2 = wc * lc * (_THR / (1.0 + _THR))
        ix = (jnp.minimum(x2r[r0:r1], x2c) - jnp.maximum(x1r[r0:r1], x1c))
        iy = jnp.maximum(
            jnp.minimum(y2r[r0:r1], y2c) - jnp.maximum(y1r[r0:r1], y1c), 0.0)
        inter = ix * iy
        return inter - (ar2[r0:r1] + ac2)

    def lane_fold(s):
        m = s[:, 0:128]
        for k in range(128, s.shape[1], 128):
            m = jnp.maximum(m, s[:, k:k + 128])
        return m

    def body(j, m):
        return jnp.maximum(m, lane_fold(tile_s(pl.ds(j * _T, _T), 0, _T, _T)))

    m = lax.fori_loop(0, i, body, jnp.full((_T, 128), -1.0, jnp.float32))
    _H = _T // 2
    row_ids = lax.broadcasted_iota(jnp.int32, (_H, 1), 0)
    col_ids = lax.broadcasted_iota(jnp.int32, (1, _H), 1)
    tri = col_ids < row_ids
    sa = tile_s(pl.ds(i * _T, _H), 0, _H, _H)
    m0 = lane_fold(jnp.where(tri, sa, -1.0))
    sb = tile_s(pl.ds(i * _T, _H), _H, _T, _H)
    sc = tile_s(pl.ds(i * _T + _H, _H), _H, _T, _H)
    m1 = jnp.maximum(lane_fold(sb), lane_fold(jnp.where(tri, sc, -1.0)))
    m = jnp.maximum(m, jnp.concatenate([m0, m1], axis=0))
    keep = (jnp.max(m, axis=1, keepdims=True) <= 0.0).astype(jnp.float32)
    row8 = jnp.concatenate(
        [cxr, cyr, rowv(czc_ref), wr, lr, rowv(hc_ref), rowv(sc_ref),
         jnp.zeros((_T, 1), jnp.float32)], axis=1)
    out_ref[...] = row8 * keep


_col_spec = pl.BlockSpec((1, _KP), lambda i: (0, 0))

_nms_call = pl.pallas_call(
    _nms_body,
    grid=(_NSTRIP,),
    in_specs=[_col_spec] * 7,
    out_specs=pl.BlockSpec((_T, 8), lambda i: (i, 0)),
    out_shape=jax.ShapeDtypeStruct((_KP, 8), jnp.float32),
)


@jax.jit
def kernel(boxes, scores):
    _, top_idx = lax.top_k(scores, _KP)
    boxes_t = boxes.T
    cols = [boxes_t[c] for c in range(6)] + [scores]
    g = _sc_gather(top_idx, *cols)
    cvecs = [v[None, :] for v in g]
    out = _nms_call(*cvecs)
    return out[:_K, :7]
